# Initial kernel scaffold; baseline (speedup 1.0000x reference)
import functools

import jax
import jax.numpy as jnp
from jax import lax
from jax.experimental import pallas as pl
from jax.experimental.pallas import tpu as pltpu
from jax.experimental.pallas import tpu_sc as plsc

N = 10000
E = 320000
D = 128
A = 16
B = 8
T = 16
H = 2
INV_AVG = 1.0 / 32.0

NTILES = 16
EDGES_PER_TILE = E // NTILES
K = 80
NB = EDGES_PER_TILE // K
ACC_ROWS = 10240


def _edge_coeff_body(ee_ref, ea_ref, we1_ref, we2_ref, p_ref, out_ref):
    ee = ee_ref[...]
    hid = jnp.dot(ee, we1_ref[...], preferred_element_type=jnp.float32)
    hid = hid * jax.nn.sigmoid(hid)
    ew = jnp.dot(hid, we2_ref[...], preferred_element_type=jnp.float32)
    ea = ea_ref[...]
    for h in range(H):
        gate = jnp.dot(ea, p_ref[h], preferred_element_type=jnp.float32)
        out_ref[h] = ew * gate * INV_AVG


def _edge_coeff(ee, ea, We1, We2, P):
    blk = 2000
    grid = (E // blk,)
    return pl.pallas_call(
        _edge_coeff_body,
        grid=grid,
        in_specs=[
            pl.BlockSpec((blk, B), lambda e: (e, 0)),
            pl.BlockSpec((blk, A), lambda e: (e, 0)),
            pl.BlockSpec((B, B), lambda e: (0, 0)),
            pl.BlockSpec((B, D), lambda e: (0, 0)),
            pl.BlockSpec((H, A, D), lambda e: (0, 0, 0)),
        ],
        out_specs=pl.BlockSpec((H, blk, D), lambda e: (0, e, 0)),
        out_shape=jax.ShapeDtypeStruct((H, E, D), jnp.float32),
    )(ee, ea, We1, We2, P)


def _x1_body(x_ref, w_ref, out_ref):
    out_ref[...] = jnp.dot(x_ref[...], w_ref[...],
                           preferred_element_type=jnp.float32)


def _x1(x, W1):
    blk = 1000
    return pl.pallas_call(
        _x1_body,
        grid=(N // blk,),
        in_specs=[
            pl.BlockSpec((blk, D), lambda i: (i, 0)),
            pl.BlockSpec((D, D), lambda i: (0, 0)),
        ],
        out_specs=pl.BlockSpec((blk, D), lambda i: (i, 0)),
        out_shape=jax.ShapeDtypeStruct((N, D), jnp.float32),
    )(x, W1)


def _sc_body(x1_hbm, c_hbm, src_hbm, dst_hbm, out_hbm,
             src_v, dst_v, g_v, c_v, stage_v, acc_sh, sem):
    h = lax.axis_index("c")
    s = lax.axis_index("s")

    def _zrow(i, carry):
        for j in range(D // 16):
            stage_v[i, pl.ds(j * 16, 16)] = jnp.zeros((16,), jnp.float32)
        return carry
    lax.fori_loop(0, 128, _zrow, 0)
    for kk in range(ACC_ROWS // NTILES // 128):
        pltpu.sync_copy(stage_v, acc_sh.at[pl.ds((s * 5 + kk) * 128, 128)])
    plsc.subcore_barrier()

    e_base = s * EDGES_PER_TILE

    def _batch(b, carry):
        e0 = e_base + b * K
        pltpu.sync_copy(src_hbm.at[pl.ds(e0, K)], src_v)
        pltpu.sync_copy(dst_hbm.at[pl.ds(e0, K)], dst_v)
        pltpu.async_copy(x1_hbm.at[src_v], g_v, sem).wait()
        pltpu.sync_copy(c_hbm.at[pl.ds(h * E + e0, K)], c_v)

        def _mrow(i, c2):
            for j in range(D // 16):
                sl = pl.ds(j * 16, 16)
                g_v[i, sl] = g_v[i, sl] * c_v[i, sl]
            return c2
        lax.fori_loop(0, K, _mrow, 0)
        pltpu.sync_copy(g_v, acc_sh.at[dst_v], add=True)
        return carry

    lax.fori_loop(0, NB, _batch, 0)
    plsc.subcore_barrier()

    rows = N // NTILES
    pltpu.sync_copy(acc_sh.at[pl.ds(s * rows, rows)],
                    out_hbm.at[pl.ds(h * N + s * rows, rows)])


def _sc_scatter(x1, c2d, src, dst):
    mesh = plsc.VectorSubcoreMesh(core_axis_name="c", subcore_axis_name="s")
    f = pl.kernel(
        _sc_body,
        out_type=jax.ShapeDtypeStruct((H * N, D), jnp.float32),
        mesh=mesh,
        scratch_types=[
            pltpu.VMEM((K,), jnp.int32),
            pltpu.VMEM((K,), jnp.int32),
            pltpu.VMEM((K, D), jnp.float32),
            pltpu.VMEM((K, D), jnp.float32),
            pltpu.VMEM((128, D), jnp.float32),
            pltpu.VMEM_SHARED((ACC_ROWS, D), jnp.float32),
            pltpu.SemaphoreType.DMA,
        ],
    )
    return f(x1, c2d, src, dst)


def _final_body(x_ref, na_ref, agg_ref, w2_ref, wsc_ref, out_ref):
    x = x_ref[...]
    na = na_ref[...]
    for h in range(H):
        r = jnp.dot(agg_ref[h], w2_ref[h], preferred_element_type=jnp.float32)
        for t in range(T):
            r = r + na[:, t:t + 1] * jnp.dot(
                x, wsc_ref[h, t], preferred_element_type=jnp.float32)
        out_ref[h] = r * jax.nn.sigmoid(r)


def _final(x, na, agg, W2, WscT):
    blk = 1000
    return pl.pallas_call(
        _final_body,
        grid=(N // blk,),
        in_specs=[
            pl.BlockSpec((blk, D), lambda i: (i, 0)),
            pl.BlockSpec((blk, T), lambda i: (i, 0)),
            pl.BlockSpec((H, blk, D), lambda i: (0, i, 0)),
            pl.BlockSpec((H, D, D), lambda i: (0, 0, 0)),
            pl.BlockSpec((H, T, D, D), lambda i: (0, 0, 0, 0)),
        ],
        out_specs=pl.BlockSpec((H, blk, D), lambda i: (0, i, 0)),
        out_shape=jax.ShapeDtypeStruct((H, N, D), jnp.float32),
    )(x, na, agg, W2, WscT)


def kernel(x, node_attrs, edge_embedding, edge_attr, edge_index,
           W1, We1, We2, P, W2, Wsc):
    src = edge_index[0]
    dst = edge_index[1]
    c = _edge_coeff(edge_embedding, edge_attr, We1, We2, P)
    x1 = _x1(x, W1)
    agg2d = _sc_scatter(x1, c.reshape(H * E, D), src, dst)
    agg = agg2d.reshape(H, N, D)
    WscT = jnp.transpose(Wsc, (0, 2, 1, 3))
    return _final(x, node_attrs, agg, W2, WscT)


# speedup vs baseline: 1.8038x; 1.8038x over previous
import functools

import jax
import jax.numpy as jnp
from jax import lax
from jax.experimental import pallas as pl
from jax.experimental.pallas import tpu as pltpu
from jax.experimental.pallas import tpu_sc as plsc

N = 10000
E = 320000
D = 128
A = 16
B = 8
T = 16
H = 2
INV_AVG = 1.0 / 32.0

NTILES = 16
EDGES_PER_TILE = E // NTILES
K = 80
NB = EDGES_PER_TILE // K
ACC_ROWS = 10240


def _edge_coeff_body(ee_ref, ea_ref, we1_ref, we2_ref, p_ref, out_ref):
    ee = ee_ref[...]
    hid = jnp.dot(ee, we1_ref[...], preferred_element_type=jnp.float32)
    hid = hid * jax.nn.sigmoid(hid)
    ew = jnp.dot(hid, we2_ref[...], preferred_element_type=jnp.float32)
    ea = ea_ref[...]
    for h in range(H):
        gate = jnp.dot(ea, p_ref[h], preferred_element_type=jnp.float32)
        out_ref[h] = ew * gate * INV_AVG


def _edge_coeff(ee, ea, We1, We2, P):
    blk = 2000
    grid = (E // blk,)
    return pl.pallas_call(
        _edge_coeff_body,
        grid=grid,
        in_specs=[
            pl.BlockSpec((blk, B), lambda e: (e, 0)),
            pl.BlockSpec((blk, A), lambda e: (e, 0)),
            pl.BlockSpec((B, B), lambda e: (0, 0)),
            pl.BlockSpec((B, D), lambda e: (0, 0)),
            pl.BlockSpec((H, A, D), lambda e: (0, 0, 0)),
        ],
        out_specs=pl.BlockSpec((H, blk, D), lambda e: (0, e, 0)),
        out_shape=jax.ShapeDtypeStruct((H, E, D), jnp.float32),
    )(ee, ea, We1, We2, P)


def _x1_body(x_ref, w_ref, out_ref):
    out_ref[...] = jnp.dot(x_ref[...], w_ref[...],
                           preferred_element_type=jnp.float32)


def _x1(x, W1):
    blk = 1000
    return pl.pallas_call(
        _x1_body,
        grid=(N // blk,),
        in_specs=[
            pl.BlockSpec((blk, D), lambda i: (i, 0)),
            pl.BlockSpec((D, D), lambda i: (0, 0)),
        ],
        out_specs=pl.BlockSpec((blk, D), lambda i: (i, 0)),
        out_shape=jax.ShapeDtypeStruct((N, D), jnp.float32),
    )(x, W1)


def _sc_body(x1_hbm, c_hbm, src_hbm, dst_hbm, out_hbm,
             src_v, dst_v, g_v, c_v, stage_v, acc_sh, sem):
    h = lax.axis_index("c")
    s = lax.axis_index("s")

    def _zrow(i, carry):
        for j in range(D // 16):
            stage_v[i, pl.ds(j * 16, 16)] = jnp.zeros((16,), jnp.float32)
        return carry
    lax.fori_loop(0, 128, _zrow, 0)
    for kk in range(ACC_ROWS // NTILES // 128):
        pltpu.sync_copy(stage_v, acc_sh.at[pl.ds((s * 5 + kk) * 128, 128)])
    plsc.subcore_barrier()

    e_base = s * EDGES_PER_TILE

    def _batch(b, carry):
        e0 = e_base + b * K
        pltpu.sync_copy(src_hbm.at[pl.ds(e0, K)], src_v)
        pltpu.sync_copy(dst_hbm.at[pl.ds(e0, K)], dst_v)
        pltpu.async_copy(x1_hbm.at[src_v], g_v, sem).wait()
        pltpu.sync_copy(c_hbm.at[pl.ds(h * E + e0, K)], c_v)

        def _mrow(i, c2):
            for j in range(D // 16):
                sl = pl.ds(j * 16, 16)
                g_v[i, sl] = g_v[i, sl] * c_v[i, sl]
            return c2
        lax.fori_loop(0, K, _mrow, 0)
        pltpu.sync_copy(g_v, acc_sh.at[dst_v], add=True)
        return carry

    lax.fori_loop(0, NB, _batch, 0)
    plsc.subcore_barrier()

    for kk in range(5):
        cid = s + NTILES * kk
        @pl.when(cid < 78)
        def _copy():
            r0 = cid * 128
            pltpu.sync_copy(acc_sh.at[pl.ds(r0, 128)],
                            out_hbm.at[pl.ds(h * N + r0, 128)])

    @pl.when(s == 14)
    def _copy_tail():
        pltpu.sync_copy(acc_sh.at[pl.ds(9984, 16)],
                        out_hbm.at[pl.ds(h * N + 9984, 16)])


def _sc_scatter(x1, c2d, src, dst):
    mesh = plsc.VectorSubcoreMesh(core_axis_name="c", subcore_axis_name="s")
    f = pl.kernel(
        _sc_body,
        out_type=jax.ShapeDtypeStruct((H * N, D), jnp.float32),
        mesh=mesh,
        scratch_types=[
            pltpu.VMEM((K,), jnp.int32),
            pltpu.VMEM((K,), jnp.int32),
            pltpu.VMEM((K, D), jnp.float32),
            pltpu.VMEM((K, D), jnp.float32),
            pltpu.VMEM((128, D), jnp.float32),
            pltpu.VMEM_SHARED((ACC_ROWS, D), jnp.float32),
            pltpu.SemaphoreType.DMA,
        ],
    )
    return f(x1, c2d, src, dst)


def _final_body(x_ref, na_ref, agg_ref, w2_ref, wsc_ref, out_ref):
    x = x_ref[...]
    na = na_ref[...]
    for h in range(H):
        r = jnp.dot(agg_ref[h], w2_ref[h], preferred_element_type=jnp.float32)
        for t in range(T):
            r = r + na[:, t:t + 1] * jnp.dot(
                x, wsc_ref[h, t], preferred_element_type=jnp.float32)
        out_ref[h] = r * jax.nn.sigmoid(r)


def _final(x, na, agg, W2, WscT):
    blk = 1000
    return pl.pallas_call(
        _final_body,
        grid=(N // blk,),
        in_specs=[
            pl.BlockSpec((blk, D), lambda i: (i, 0)),
            pl.BlockSpec((blk, T), lambda i: (i, 0)),
            pl.BlockSpec((H, blk, D), lambda i: (0, i, 0)),
            pl.BlockSpec((H, D, D), lambda i: (0, 0, 0)),
            pl.BlockSpec((H, T, D, D), lambda i: (0, 0, 0, 0)),
        ],
        out_specs=pl.BlockSpec((H, blk, D), lambda i: (0, i, 0)),
        out_shape=jax.ShapeDtypeStruct((H, N, D), jnp.float32),
    )(x, na, agg, W2, WscT)


def kernel(x, node_attrs, edge_embedding, edge_attr, edge_index,
           W1, We1, We2, P, W2, Wsc):
    src = edge_index[0]
    dst = edge_index[1]
    c = _edge_coeff(edge_embedding, edge_attr, We1, We2, P)
    x1 = _x1(x, W1)
    agg2d = _sc_scatter(x1, c.reshape(H * E, D), src, dst)
    agg = agg2d.reshape(H, N, D)
    WscT = jnp.transpose(Wsc, (0, 2, 1, 3))
    return _final(x, node_attrs, agg, W2, WscT)


# speedup vs baseline: 1.8882x; 1.0468x over previous
import functools

import jax
import jax.numpy as jnp
from jax import lax
from jax.experimental import pallas as pl
from jax.experimental.pallas import tpu as pltpu
from jax.experimental.pallas import tpu_sc as plsc

N = 10000
E = 320000
D = 128
A = 16
B = 8
T = 16
H = 2
INV_AVG = 1.0 / 32.0

NTILES = 16
EDGES_PER_TILE = E // NTILES
K = 80
NB = EDGES_PER_TILE // K
ACC_ROWS = 10240


def _edge_coeff_body(ee_ref, ea_ref, we1_ref, we2_ref, p_ref, out_ref):
    ee = ee_ref[...]
    hid = jnp.dot(ee, we1_ref[...], preferred_element_type=jnp.float32)
    hid = hid * jax.nn.sigmoid(hid)
    ew = jnp.dot(hid, we2_ref[...], preferred_element_type=jnp.float32)
    ea = ea_ref[...]
    for h in range(H):
        gate = jnp.dot(ea, p_ref[h], preferred_element_type=jnp.float32)
        out_ref[h] = ew * gate * INV_AVG


def _edge_coeff(ee, ea, We1, We2, P):
    blk = 2000
    grid = (E // blk,)
    return pl.pallas_call(
        _edge_coeff_body,
        grid=grid,
        in_specs=[
            pl.BlockSpec((blk, B), lambda e: (e, 0)),
            pl.BlockSpec((blk, A), lambda e: (e, 0)),
            pl.BlockSpec((B, B), lambda e: (0, 0)),
            pl.BlockSpec((B, D), lambda e: (0, 0)),
            pl.BlockSpec((H, A, D), lambda e: (0, 0, 0)),
        ],
        out_specs=pl.BlockSpec((H, blk, D), lambda e: (0, e, 0)),
        out_shape=jax.ShapeDtypeStruct((H, E, D), jnp.float32),
    )(ee, ea, We1, We2, P)


def _x1_body(x_ref, w_ref, out_ref):
    out_ref[...] = jnp.dot(x_ref[...], w_ref[...],
                           preferred_element_type=jnp.float32)


def _x1(x, W1):
    blk = 1000
    return pl.pallas_call(
        _x1_body,
        grid=(N // blk,),
        in_specs=[
            pl.BlockSpec((blk, D), lambda i: (i, 0)),
            pl.BlockSpec((D, D), lambda i: (0, 0)),
        ],
        out_specs=pl.BlockSpec((blk, D), lambda i: (i, 0)),
        out_shape=jax.ShapeDtypeStruct((N, D), jnp.float32),
    )(x, W1)


def _sc_body(x1_hbm, c_hbm, src_hbm, dst_hbm, out_hbm,
             s0, s1, d0, d1, g0, g1, c0, c1, acc_sh,
             gsem0, gsem1, csem0, csem1, isem0, isem1):
    h = lax.axis_index("c")
    s = lax.axis_index("s")
    SV = (s0, s1)
    DV = (d0, d1)
    G = (g0, g1)
    C = (c0, c1)
    GS = (gsem0, gsem1)
    CS = (csem0, csem1)
    IS = (isem0, isem1)

    def _zrow(i, carry):
        for j in range(D // 16):
            c0[i, pl.ds(j * 16, 16)] = jnp.zeros((16,), jnp.float32)
        return carry
    lax.fori_loop(0, K, _zrow, 0)
    for kk in range(8):
        pltpu.sync_copy(c0, acc_sh.at[pl.ds(s * 640 + kk * K, K)])
    plsc.subcore_barrier()

    e_base = s * EDGES_PER_TILE

    def _issue_idx(b, buf):
        pltpu.async_copy(src_hbm.at[pl.ds(e_base + b * K, K)], SV[buf], IS[buf])
        pltpu.async_copy(dst_hbm.at[pl.ds(e_base + b * K, K)], DV[buf], IS[buf])

    def _wait_idx(b, buf):
        pltpu.make_async_copy(src_hbm.at[pl.ds(e_base + b * K, K)],
                              SV[buf], IS[buf]).wait()
        pltpu.make_async_copy(dst_hbm.at[pl.ds(e_base + b * K, K)],
                              DV[buf], IS[buf]).wait()

    def _issue_main(b, buf):
        pltpu.async_copy(x1_hbm.at[SV[buf]], G[buf], GS[buf])
        pltpu.async_copy(c_hbm.at[pl.ds(h * E + e_base + b * K, K)],
                         C[buf], CS[buf])

    _issue_idx(0, 0)
    _issue_idx(1, 1)
    _wait_idx(0, 0)
    _issue_main(0, 0)

    def _pair(t, carry):
        for buf in range(2):
            b = 2 * t + buf
            nbuf = 1 - buf

            @pl.when(b + 1 < NB)
            def _nxt():
                _wait_idx(b + 1, nbuf)
                _issue_main(b + 1, nbuf)

            pltpu.make_async_copy(x1_hbm.at[SV[buf]], G[buf], GS[buf]).wait()
            pltpu.make_async_copy(c_hbm.at[pl.ds(h * E + e_base + b * K, K)],
                                  C[buf], CS[buf]).wait()

            def _mrow(i, c2):
                for j in range(D // 16):
                    sl = pl.ds(j * 16, 16)
                    G[buf][i, sl] = G[buf][i, sl] * C[buf][i, sl]
                return c2
            lax.fori_loop(0, K, _mrow, 0, unroll=2)

            pltpu.sync_copy(G[buf], acc_sh.at[DV[buf]], add=True)

            @pl.when(b + 2 < NB)
            def _nxt_idx():
                _issue_idx(b + 2, buf)
        return carry

    lax.fori_loop(0, NB // 2, _pair, 0)
    plsc.subcore_barrier()

    for kk in range(5):
        cid = s + NTILES * kk
        @pl.when(cid < 78)
        def _copy():
            r0 = cid * 128
            pltpu.sync_copy(acc_sh.at[pl.ds(r0, 128)],
                            out_hbm.at[pl.ds(h * N + r0, 128)])

    @pl.when(s == 14)
    def _copy_tail():
        pltpu.sync_copy(acc_sh.at[pl.ds(9984, 16)],
                        out_hbm.at[pl.ds(h * N + 9984, 16)])


def _sc_scatter(x1, c2d, src, dst):
    mesh = plsc.VectorSubcoreMesh(core_axis_name="c", subcore_axis_name="s")
    f = pl.kernel(
        _sc_body,
        out_type=jax.ShapeDtypeStruct((H * N, D), jnp.float32),
        mesh=mesh,
        scratch_types=[
            pltpu.VMEM((K,), jnp.int32),
            pltpu.VMEM((K,), jnp.int32),
            pltpu.VMEM((K,), jnp.int32),
            pltpu.VMEM((K,), jnp.int32),
            pltpu.VMEM((K, D), jnp.float32),
            pltpu.VMEM((K, D), jnp.float32),
            pltpu.VMEM((K, D), jnp.float32),
            pltpu.VMEM((K, D), jnp.float32),
            pltpu.VMEM_SHARED((ACC_ROWS, D), jnp.float32),
            pltpu.SemaphoreType.DMA,
            pltpu.SemaphoreType.DMA,
            pltpu.SemaphoreType.DMA,
            pltpu.SemaphoreType.DMA,
            pltpu.SemaphoreType.DMA,
            pltpu.SemaphoreType.DMA,
        ],
    )
    return f(x1, c2d, src, dst)


def _final_body(x_ref, na_ref, agg_ref, w2_ref, wsc_ref, out_ref):
    x = x_ref[...]
    na = na_ref[...]
    for h in range(H):
        r = jnp.dot(agg_ref[h], w2_ref[h], preferred_element_type=jnp.float32)
        for t in range(T):
            r = r + na[:, t:t + 1] * jnp.dot(
                x, wsc_ref[h, t], preferred_element_type=jnp.float32)
        out_ref[h] = r * jax.nn.sigmoid(r)


def _final(x, na, agg, W2, WscT):
    blk = 1000
    return pl.pallas_call(
        _final_body,
        grid=(N // blk,),
        in_specs=[
            pl.BlockSpec((blk, D), lambda i: (i, 0)),
            pl.BlockSpec((blk, T), lambda i: (i, 0)),
            pl.BlockSpec((H, blk, D), lambda i: (0, i, 0)),
            pl.BlockSpec((H, D, D), lambda i: (0, 0, 0)),
            pl.BlockSpec((H, T, D, D), lambda i: (0, 0, 0, 0)),
        ],
        out_specs=pl.BlockSpec((H, blk, D), lambda i: (0, i, 0)),
        out_shape=jax.ShapeDtypeStruct((H, N, D), jnp.float32),
    )(x, na, agg, W2, WscT)


def kernel(x, node_attrs, edge_embedding, edge_attr, edge_index,
           W1, We1, We2, P, W2, Wsc):
    src = edge_index[0]
    dst = edge_index[1]
    c = _edge_coeff(edge_embedding, edge_attr, We1, We2, P)
    x1 = _x1(x, W1)
    agg2d = _sc_scatter(x1, c.reshape(H * E, D), src, dst)
    agg = agg2d.reshape(H, N, D)
    WscT = jnp.transpose(Wsc, (0, 2, 1, 3))
    return _final(x, node_attrs, agg, W2, WscT)
